# Initial kernel scaffold; baseline (speedup 1.0000x reference)
import jax
import jax.numpy as jnp
from jax.experimental import pallas as pl


def kernel(cate_emb_w, scene_emb_w, agg_w, agg_b, cids, cate_scene_pad, c_cate_pad):
    raise NotImplementedError("write your pallas kernel here")


# speedup vs baseline: 25.9773x; 25.9773x over previous
import functools

import jax
import jax.numpy as jnp
from jax import lax
from jax.experimental import pallas as pl
from jax.experimental.pallas import tpu as pltpu
from jax.experimental.pallas import tpu_sc as plsc

F32 = jnp.float32
I32 = jnp.int32

NUM_CORES = 2
NUM_SUBCORES = 16
NW = NUM_CORES * NUM_SUBCORES
LANES = 16
EPS = 1e-10
GC = 128


def _wid():
    return lax.axis_index("s") * NUM_CORES + lax.axis_index("c")


def _full(v):
    return jnp.full((LANES,), v, I32)


def _make_k1(NP, SV, D, LS, CHUNK, B1):
    mesh = plsc.VectorSubcoreMesh(core_axis_name="c", subcore_axis_name="s")

    @functools.partial(
        pl.kernel,
        out_type=(jax.ShapeDtypeStruct((NP, D), F32),
                  jax.ShapeDtypeStruct((NP, D), F32)),
        mesh=mesh,
        compiler_params=pltpu.CompilerParams(needs_layout_passes=False, use_tc_tiling_on_sc=False),
        scratch_types=[
            pltpu.VMEM((SV, D), F32),
            pltpu.VMEM((B1, LS), I32),
            pltpu.VMEM((B1, D), F32),
            pltpu.VMEM((B1, D), F32),
        ],
    )
    def k1(scene_hbm, csp_hbm, s_hbm, rn_hbm, scene_v, idx_v, s_v, rn_v):
        wid = _wid()
        pltpu.sync_copy(scene_hbm, scene_v)
        iota = lax.iota(I32, LANES)

        def do_block(blk, carry):
            base = wid * CHUNK + blk * B1
            pltpu.sync_copy(csp_hbm.at[pl.ds(base, B1)], idx_v)

            def grp(g, c2):
                rows = g * LANES + iota
                cols = [plsc.load_gather(idx_v, [rows, _full(s)])
                        for s in range(LS)]
                sd = []
                for d in range(D):
                    dd = _full(d)
                    acc = plsc.load_gather(scene_v, [cols[0], dd])
                    for s in range(1, LS):
                        acc = acc + plsc.load_gather(scene_v, [cols[s], dd])
                    sd.append(acc)
                n2 = sd[0] * sd[0]
                for d in range(1, D):
                    n2 = n2 + sd[d] * sd[d]
                inv = 1.0 / (n2 + EPS)
                for d in range(D):
                    dd = _full(d)
                    plsc.store_scatter(s_v, [rows, dd], sd[d])
                    plsc.store_scatter(rn_v, [rows, dd], sd[d] * inv)
                return c2

            lax.fori_loop(0, B1 // LANES, grp, 0)
            pltpu.sync_copy(s_v, s_hbm.at[pl.ds(base, B1)])
            pltpu.sync_copy(rn_v, rn_hbm.at[pl.ds(base, B1)])
            return carry

        lax.fori_loop(0, CHUNK // B1, do_block, 0)

    return k1


def _make_k2(NP, D, L, NV1, CHUNK, B2):
    IDXN = B2 * L
    mesh = plsc.VectorSubcoreMesh(core_axis_name="c", subcore_axis_name="s")

    @functools.partial(
        pl.kernel,
        out_type=jax.ShapeDtypeStruct((NP, D), F32),
        mesh=mesh,
        compiler_params=pltpu.CompilerParams(needs_layout_passes=False, use_tc_tiling_on_sc=False),
        scratch_types=[
            pltpu.VMEM((IDXN,), I32),
            pltpu.VMEM((IDXN, D), F32),
            pltpu.VMEM((IDXN, D), F32),
            pltpu.VMEM((B2, D), F32),
            pltpu.VMEM((B2, D), F32),
            pltpu.VMEM((B2, D), F32),
            pltpu.SemaphoreType.DMA,
        ],
    )
    def k2(cate_hbm, rn_hbm, ccp_hbm, agg_hbm,
           idx_v, ngr_v, ngc_v, rnself_v, cself_v, agg_v, sem):
        wid = _wid()
        iota = lax.iota(I32, LANES)

        def do_block(blk, carry):
            base = wid * CHUNK + blk * B2
            pltpu.sync_copy(ccp_hbm.at[pl.ds(base * L, IDXN)], idx_v)
            handles = []
            for j in range(IDXN // GC):
                sl = pl.ds(j * GC, GC)
                handles.append(pltpu.async_copy(
                    rn_hbm.at[idx_v.at[sl]], ngr_v.at[sl], sem))
                handles.append(pltpu.async_copy(
                    cate_hbm.at[idx_v.at[sl]], ngc_v.at[sl], sem))
            pltpu.sync_copy(rn_hbm.at[pl.ds(base, B2)], rnself_v)
            pltpu.sync_copy(cate_hbm.at[pl.ds(base, B2)], cself_v)
            for h in handles:
                h.wait()

            def grp(g, c2):
                rows = g * LANES + iota
                gid = base + rows
                rnT = [plsc.load_gather(rnself_v, [rows, _full(d)])
                       for d in range(D)]
                sim0 = rnT[0] * rnT[0]
                for d in range(1, D):
                    sim0 = sim0 + rnT[d] * rnT[d]
                e0 = jnp.where(gid < NV1, jnp.exp(sim0), 0.0)
                tot = e0
                aggacc = [e0 * plsc.load_gather(cself_v, [rows, _full(d)])
                          for d in range(D)]
                for l in range(L):
                    nrows = rows * L + l
                    vals = plsc.load_gather(idx_v, [nrows])
                    sim = rnT[0] * plsc.load_gather(ngr_v, [nrows, _full(0)])
                    for d in range(1, D):
                        sim = sim + rnT[d] * plsc.load_gather(
                            ngr_v, [nrows, _full(d)])
                    e = jnp.where(vals < NV1, jnp.exp(sim), 0.0)
                    tot = tot + e
                    for d in range(D):
                        aggacc[d] = aggacc[d] + e * plsc.load_gather(
                            ngc_v, [nrows, _full(d)])
                inv = 1.0 / (tot + EPS)
                for d in range(D):
                    plsc.store_scatter(agg_v, [rows, _full(d)],
                                       aggacc[d] * inv)
                return c2

            lax.fori_loop(0, B2 // LANES, grp, 0)
            pltpu.sync_copy(agg_v, agg_hbm.at[pl.ds(base, B2)])
            return carry

        lax.fori_loop(0, CHUNK // B2, do_block, 0)

    return k2


def _k3_body(s_ref, a_ref, w1_ref, w2_ref, b_ref, o_ref):
    z = jnp.dot(s_ref[...], w1_ref[...], preferred_element_type=F32)
    z = z + jnp.dot(a_ref[...], w2_ref[...], preferred_element_type=F32)
    z = z + b_ref[...]
    o_ref[...] = jnp.where(z > 0, z, jnp.exp(z) - 1.0)


def kernel(cate_emb_w, scene_emb_w, agg_w, agg_b, cids, cate_scene_pad,
           c_cate_pad):
    N, D = cate_emb_w.shape
    SV = scene_emb_w.shape[0]
    L = c_cate_pad.shape[1]
    LS = cate_scene_pad.shape[1]
    NV1 = N - 1

    NP = -(-N // (NW * LANES)) * (NW * LANES)
    CHUNK = NP // NW
    B1 = CHUNK // 2
    B2 = 64
    assert CHUNK % B2 == 0 and (B2 * L) % GC == 0

    cate_p = jnp.pad(cate_emb_w, ((0, NP - N), (0, 0)))
    csp_p = jnp.pad(cate_scene_pad.astype(I32), ((0, NP - N), (0, 0)))
    ccp_flat = jnp.pad(c_cate_pad.astype(I32),
                       ((0, NP - N), (0, 0))).reshape(NP * L)

    s_all, rn_all = _make_k1(NP, SV, D, LS, CHUNK, B1)(scene_emb_w, csp_p)
    agg = _make_k2(NP, D, L, NV1, CHUNK, B2)(cate_p, rn_all, ccp_flat)

    w1t = agg_w[:, :D].T
    w2t = agg_w[:, D:].T
    b2 = agg_b.reshape(1, D)
    BK3 = 1024
    out = pl.pallas_call(
        _k3_body,
        grid=(NP // BK3,),
        in_specs=[
            pl.BlockSpec((BK3, D), lambda i: (i, 0)),
            pl.BlockSpec((BK3, D), lambda i: (i, 0)),
            pl.BlockSpec((D, D), lambda i: (0, 0)),
            pl.BlockSpec((D, D), lambda i: (0, 0)),
            pl.BlockSpec((1, D), lambda i: (0, 0)),
        ],
        out_specs=pl.BlockSpec((BK3, D), lambda i: (i, 0)),
        out_shape=jax.ShapeDtypeStruct((NP, D), F32),
    )(s_all, agg, w1t, w2t, b2)
    return out[:N]


# speedup vs baseline: 30.1050x; 1.1589x over previous
import functools

import jax
import jax.numpy as jnp
from jax import lax
from jax.experimental import pallas as pl
from jax.experimental.pallas import tpu as pltpu
from jax.experimental.pallas import tpu_sc as plsc

F32 = jnp.float32
I32 = jnp.int32

NUM_CORES = 2
NUM_SUBCORES = 16
NW = NUM_CORES * NUM_SUBCORES
LANES = 16
EPS = 1e-10
GC = 112
DP = 17
B2 = 64

_SC_PARAMS = pltpu.CompilerParams(needs_layout_passes=False,
                                  use_tc_tiling_on_sc=False)


def _wid():
    return lax.axis_index("s") * NUM_CORES + lax.axis_index("c")


def _make_k1(NP, SV, D, LS, LSP, CHUNK, B1):
    mesh = plsc.VectorSubcoreMesh(core_axis_name="c", subcore_axis_name="s")

    @functools.partial(
        pl.kernel,
        out_type=(jax.ShapeDtypeStruct((NP, D), F32),
                  jax.ShapeDtypeStruct((NP, D), F32)),
        mesh=mesh,
        compiler_params=_SC_PARAMS,
        scratch_types=[
            pltpu.VMEM((SV, D), F32),
            pltpu.VMEM((B1 * LSP,), I32),
            pltpu.VMEM((B1, D), F32),
            pltpu.VMEM((B1, D), F32),
        ],
    )
    def k1(scene_hbm, csp_hbm, s_hbm, rn_hbm, scene_v, idx_v, s_v, rn_v):
        wid = _wid()
        pltpu.sync_copy(scene_hbm, scene_v)
        iota = lax.iota(I32, LANES)

        def do_block(blk, carry):
            base = wid * CHUNK + blk * B1
            pltpu.sync_copy(csp_hbm.at[pl.ds(base * LSP, B1 * LSP)], idx_v)

            def grp(g, c2):
                r0 = g * LANES
                ri = (r0 + iota) * LSP
                cols = [plsc.load_gather(idx_v, [ri + s]) for s in range(LS)]
                for k in range(LANES):
                    srow = scene_v[cols[0][k]]
                    for s in range(1, LS):
                        srow = srow + scene_v[cols[s][k]]
                    n2 = jnp.sum(srow * srow)
                    s_v[r0 + k] = srow
                    rn_v[r0 + k] = srow / (n2 + EPS)
                return c2

            lax.fori_loop(0, B1 // LANES, grp, 0)
            pltpu.sync_copy(s_v, s_hbm.at[pl.ds(base, B1)])
            pltpu.sync_copy(rn_v, rn_hbm.at[pl.ds(base, B1)])
            return carry

        lax.fori_loop(0, CHUNK // B1, do_block, 0)

    return k1


def _make_k2(NP, D, LP, NV1, CHUNK):
    IDXN = B2 * LP
    mesh = plsc.VectorSubcoreMesh(core_axis_name="c", subcore_axis_name="s")

    @functools.partial(
        pl.kernel,
        out_type=jax.ShapeDtypeStruct((NP * DP,), F32),
        mesh=mesh,
        compiler_params=_SC_PARAMS,
        scratch_types=[
            pltpu.VMEM((IDXN,), I32),
            pltpu.VMEM((IDXN, D), F32),
            pltpu.VMEM((IDXN, D), F32),
            pltpu.VMEM((IDXN * DP,), F32),
            pltpu.VMEM((IDXN * DP,), F32),
            pltpu.VMEM((B2 * DP,), F32),
            pltpu.SemaphoreType.DMA,
        ],
    )
    def k2(cate_hbm, rn_hbm, part_hbm, agg_hbm,
           idx_v, ngr_s, ngc_s, ngr_f, ngc_f, agg_v, sem):
        wid = _wid()
        iota = lax.iota(I32, LANES)

        def do_block(blk, carry):
            base = wid * CHUNK + blk * B2
            pltpu.sync_copy(part_hbm.at[pl.ds(base * LP, IDXN)], idx_v)
            handles = []
            for j in range(IDXN // GC):
                sl = pl.ds(j * GC, GC)
                handles.append(pltpu.async_copy(
                    rn_hbm.at[idx_v.at[sl]], ngr_s.at[sl], sem))
                handles.append(pltpu.async_copy(
                    cate_hbm.at[idx_v.at[sl]], ngc_s.at[sl], sem))
            for h in handles:
                h.wait()

            def trow(r, addr):
                plsc.store_scatter(ngr_f, [addr], ngr_s[r])
                plsc.store_scatter(ngc_f, [addr], ngc_s[r])
                return addr + DP

            lax.fori_loop(0, IDXN, trow, iota)

            def grp(g, c2):
                nr0 = (g * LANES + iota) * LP
                b17 = nr0 * DP
                rnT = [plsc.load_gather(ngr_f, [b17 + d]) for d in range(D)]
                tot = None
                aggacc = None
                for l in range(LP):
                    el = b17 + (l * DP)
                    vals = plsc.load_gather(idx_v, [nr0 + l])
                    sim = rnT[0] * plsc.load_gather(ngr_f, [el])
                    for d in range(1, D):
                        sim = sim + rnT[d] * plsc.load_gather(ngr_f, [el + d])
                    e = jnp.where(vals < NV1, jnp.exp(sim), 0.0)
                    tot = e if tot is None else tot + e
                    news = [e * plsc.load_gather(ngc_f, [el + d])
                            for d in range(D)]
                    aggacc = news if aggacc is None else [
                        a + b for a, b in zip(aggacc, news)]
                inv = 1.0 / (tot + EPS)
                ro = (g * LANES + iota) * DP
                for d in range(D):
                    plsc.store_scatter(agg_v, [ro + d], aggacc[d] * inv)
                return c2

            lax.fori_loop(0, B2 // LANES, grp, 0)
            pltpu.sync_copy(agg_v, agg_hbm.at[pl.ds(base * DP, B2 * DP)])
            return carry

        lax.fori_loop(0, CHUNK // B2, do_block, 0)

    return k2


def _k3_body(s_ref, a_ref, w1_ref, w2_ref, b_ref, o_ref):
    z = jnp.dot(s_ref[...], w1_ref[...], preferred_element_type=F32)
    z = z + jnp.dot(a_ref[...], w2_ref[...], preferred_element_type=F32)
    z = z + b_ref[...]
    o_ref[...] = jnp.where(z > 0, z, jnp.exp(z) - 1.0)


def kernel(cate_emb_w, scene_emb_w, agg_w, agg_b, cids, cate_scene_pad,
           c_cate_pad):
    N, D = cate_emb_w.shape
    SV = scene_emb_w.shape[0]
    L = c_cate_pad.shape[1]
    LP = L + 1
    LS = cate_scene_pad.shape[1]
    LSP = LS + 1
    NV1 = N - 1

    ALIGN = NW * B2
    NP = -(-N // ALIGN) * ALIGN
    CHUNK = NP // NW
    B1 = CHUNK // 2
    assert B1 % LANES == 0 and (B2 * LP) % GC == 0

    cate_p = jnp.pad(cate_emb_w, ((0, NP - N), (0, 0)))
    csp_flat = jnp.pad(cate_scene_pad.astype(I32),
                       ((0, NP - N), (0, LSP - LS))).reshape(NP * LSP)
    ccp_p = jnp.pad(c_cate_pad.astype(I32), ((0, NP - N), (0, 0)))
    part_flat = jnp.concatenate(
        [jnp.arange(NP, dtype=I32)[:, None], ccp_p], axis=1).reshape(NP * LP)

    s_all, rn_all = _make_k1(NP, SV, D, LS, LSP, CHUNK, B1)(
        scene_emb_w, csp_flat)
    agg_flat = _make_k2(NP, D, LP, NV1, CHUNK)(cate_p, rn_all, part_flat)

    a_in = agg_flat.reshape(NP, DP)[:, :D]
    w1t = agg_w[:, :D].T
    w2t = agg_w[:, D:].T
    b2 = agg_b.reshape(1, D)
    BK3 = 1024
    out = pl.pallas_call(
        _k3_body,
        grid=(NP // BK3,),
        in_specs=[
            pl.BlockSpec((BK3, D), lambda i: (i, 0)),
            pl.BlockSpec((BK3, D), lambda i: (i, 0)),
            pl.BlockSpec((D, D), lambda i: (0, 0)),
            pl.BlockSpec((D, D), lambda i: (0, 0)),
            pl.BlockSpec((1, D), lambda i: (0, 0)),
        ],
        out_specs=pl.BlockSpec((BK3, D), lambda i: (i, 0)),
        out_shape=jax.ShapeDtypeStruct((NP, D), F32),
    )(s_all, a_in, w1t, w2t, b2)
    return out[:N]


# speedup vs baseline: 39.7779x; 1.3213x over previous
import functools

import jax
import jax.numpy as jnp
from jax import lax
from jax.experimental import pallas as pl
from jax.experimental.pallas import tpu as pltpu
from jax.experimental.pallas import tpu_sc as plsc

F32 = jnp.float32
I32 = jnp.int32

NUM_CORES = 2
NUM_SUBCORES = 16
NW = NUM_CORES * NUM_SUBCORES
LANES = 16
EPS = 1e-10
GC = 96
DP = 17
B2 = 32

_SC_PARAMS = pltpu.CompilerParams(needs_layout_passes=False,
                                  use_tc_tiling_on_sc=False)


def _wid():
    return lax.axis_index("s") * NUM_CORES + lax.axis_index("c")


def _make_k1(NP, SV, D, LS, LSP, CHUNK, B1):
    mesh = plsc.VectorSubcoreMesh(core_axis_name="c", subcore_axis_name="s")

    @functools.partial(
        pl.kernel,
        out_type=(jax.ShapeDtypeStruct((NP, D), F32),
                  jax.ShapeDtypeStruct((NP, D), F32)),
        mesh=mesh,
        compiler_params=_SC_PARAMS,
        scratch_types=[
            pltpu.VMEM((SV, D), F32),
            pltpu.VMEM((B1 * LSP,), I32),
            pltpu.VMEM((B1, D), F32),
            pltpu.VMEM((B1, D), F32),
        ],
    )
    def k1(scene_hbm, csp_hbm, s_hbm, rn_hbm, scene_v, idx_v, s_v, rn_v):
        wid = _wid()
        pltpu.sync_copy(scene_hbm, scene_v)
        iota = lax.iota(I32, LANES)

        def do_block(blk, carry):
            base = wid * CHUNK + blk * B1
            pltpu.sync_copy(csp_hbm.at[pl.ds(base * LSP, B1 * LSP)], idx_v)

            def grp(g, c2):
                r0 = g * LANES
                ri = (r0 + iota) * LSP
                cols = [plsc.load_gather(idx_v, [ri + s]) for s in range(LS)]
                for k in range(LANES):
                    srow = scene_v[cols[0][k]]
                    for s in range(1, LS):
                        srow = srow + scene_v[cols[s][k]]
                    n2 = jnp.sum(srow * srow)
                    s_v[r0 + k] = srow
                    rn_v[r0 + k] = srow / (n2 + EPS)
                return c2

            lax.fori_loop(0, B1 // LANES, grp, 0)
            pltpu.sync_copy(s_v, s_hbm.at[pl.ds(base, B1)])
            pltpu.sync_copy(rn_v, rn_hbm.at[pl.ds(base, B1)])
            return carry

        lax.fori_loop(0, CHUNK // B1, do_block, 0)

    return k1


def _make_k2(NP, D, LP, NV1, CHUNK):
    IDXN = B2 * LP
    mesh = plsc.VectorSubcoreMesh(core_axis_name="c", subcore_axis_name="s")

    @functools.partial(
        pl.kernel,
        out_type=jax.ShapeDtypeStruct((NP * DP,), F32),
        mesh=mesh,
        compiler_params=_SC_PARAMS,
        scratch_types=[
            pltpu.VMEM((IDXN,), I32), pltpu.VMEM((IDXN,), I32),
            pltpu.VMEM((IDXN, D), F32), pltpu.VMEM((IDXN, D), F32),
            pltpu.VMEM((IDXN, D), F32), pltpu.VMEM((IDXN, D), F32),
            pltpu.VMEM((IDXN * DP,), F32),
            pltpu.VMEM((IDXN * DP,), F32),
            pltpu.VMEM((B2 * DP,), F32), pltpu.VMEM((B2 * DP,), F32),
            pltpu.SemaphoreType.DMA, pltpu.SemaphoreType.DMA,
            pltpu.SemaphoreType.DMA, pltpu.SemaphoreType.DMA,
            pltpu.SemaphoreType.DMA, pltpu.SemaphoreType.DMA,
        ],
    )
    def k2(cate_hbm, rn_hbm, part_hbm, agg_hbm,
           idx0, idx1, str0, str1, stc0, stc1, ngr_f, ngc_f, agg0, agg1,
           sg0, sg1, si0, si1, so0, so1):
        wid = _wid()
        iota = lax.iota(I32, LANES)
        base0 = wid * CHUNK
        NBLK = CHUNK // B2
        idxs, strs, stcs = (idx0, idx1), (str0, str1), (stc0, stc1)
        aggs, sgs, sis, sos = (agg0, agg1), (sg0, sg1), (si0, si1), (so0, so1)

        def issue_gathers(b):
            for j in range(IDXN // GC):
                sl = pl.ds(j * GC, GC)
                pltpu.async_copy(rn_hbm.at[idxs[b].at[sl]],
                                 strs[b].at[sl], sgs[b])
                pltpu.async_copy(cate_hbm.at[idxs[b].at[sl]],
                                 stcs[b].at[sl], sgs[b])

        def wait_gathers(b):
            pltpu.make_async_copy(rn_hbm.at[pl.ds(0, IDXN)],
                                  strs[b], sgs[b]).wait()
            pltpu.make_async_copy(rn_hbm.at[pl.ds(0, IDXN)],
                                  stcs[b], sgs[b]).wait()

        def issue_idx(blk, b):
            pltpu.async_copy(
                part_hbm.at[pl.ds((base0 + blk * B2) * LP, IDXN)],
                idxs[b], sis[b])

        def wait_idx(b):
            pltpu.make_async_copy(part_hbm.at[pl.ds(0, IDXN)],
                                  idxs[b], sis[b]).wait()

        def issue_out(blk, b):
            pltpu.async_copy(
                aggs[b],
                agg_hbm.at[pl.ds((base0 + blk * B2) * DP, B2 * DP)], sos[b])

        def wait_out(b):
            pltpu.make_async_copy(aggs[b], agg_hbm.at[pl.ds(0, B2 * DP)],
                                  sos[b]).wait()

        TR = 8

        def transpose(b):
            ngr_s, ngc_s = strs[b], stcs[b]

            def trow(r0, addr):
                for k in range(TR):
                    r = r0 * TR + k
                    a = addr + k * DP
                    plsc.store_scatter(ngr_f, [a], ngr_s[r])
                    plsc.store_scatter(ngc_f, [a], ngc_s[r])
                return addr + TR * DP

            lax.fori_loop(0, IDXN // TR, trow, iota)

        def make_grp(idx_v, agg_v):
            def grp(g, c2):
                nr0 = (g * LANES + iota) * LP
                b17 = nr0 * DP
                sims = [None] * LP
                for d in range(D):
                    rnd = plsc.load_gather(ngr_f, [b17 + d])
                    for l in range(LP):
                        el = b17 + (l * DP + d)
                        g_ = rnd * rnd if l == 0 else rnd * plsc.load_gather(
                            ngr_f, [el])
                        sims[l] = g_ if d == 0 else sims[l] + g_
                es = []
                tot = None
                for l in range(LP):
                    vals = plsc.load_gather(idx_v, [nr0 + l])
                    e = jnp.where(vals < NV1, jnp.exp(sims[l]), 0.0)
                    es.append(e)
                    tot = e if tot is None else tot + e
                inv = 1.0 / (tot + EPS)
                ro = (g * LANES + iota) * DP
                for d in range(D):
                    prods = [es[l] * plsc.load_gather(
                        ngc_f, [b17 + (l * DP + d)]) for l in range(LP)]
                    while len(prods) > 1:
                        nxt = [a + b for a, b in zip(prods[::2], prods[1::2])]
                        if len(prods) % 2:
                            nxt[-1] = nxt[-1] + prods[-1]
                        prods = nxt
                    plsc.store_scatter(agg_v, [ro + d], prods[0] * inv)
                return c2

            return grp

        def process(blk, b):
            @pl.when(blk + 1 < NBLK)
            def _():
                wait_idx(1 - b)
                issue_gathers(1 - b)

            wait_gathers(b)
            transpose(b)

            @pl.when(blk >= 2)
            def _():
                wait_out(b)

            lax.fori_loop(0, B2 // LANES, make_grp(idxs[b], aggs[b]), 0)
            issue_out(blk, b)

            @pl.when(blk + 2 < NBLK)
            def _():
                issue_idx(blk + 2, b)

        pltpu.sync_copy(part_hbm.at[pl.ds(base0 * LP, IDXN)], idx0)
        issue_gathers(0)
        issue_idx(1, 1)

        def sb(s, carry):
            process(2 * s, 0)
            process(2 * s + 1, 1)
            return carry

        lax.fori_loop(0, NBLK // 2, sb, 0)
        wait_out(0)
        wait_out(1)

    return k2


def _k3_body(s_ref, a_ref, w1_ref, w2_ref, b_ref, o_ref):
    z = jnp.dot(s_ref[...], w1_ref[...], preferred_element_type=F32)
    z = z + jnp.dot(a_ref[...], w2_ref[...], preferred_element_type=F32)
    z = z + b_ref[...]
    o_ref[...] = jnp.where(z > 0, z, jnp.exp(z) - 1.0)


def kernel(cate_emb_w, scene_emb_w, agg_w, agg_b, cids, cate_scene_pad,
           c_cate_pad):
    N, D = cate_emb_w.shape
    SV = scene_emb_w.shape[0]
    L = c_cate_pad.shape[1]
    LP = L + 1
    LS = cate_scene_pad.shape[1]
    LSP = LS + 1
    NV1 = N - 1

    ALIGN = NW * B2
    NP = -(-N // ALIGN) * ALIGN
    CHUNK = NP // NW
    B1 = CHUNK // 2
    assert B1 % LANES == 0 and (B2 * LP) % GC == 0

    cate_p = jnp.pad(cate_emb_w, ((0, NP - N), (0, 0)))
    csp_flat = jnp.pad(cate_scene_pad.astype(I32),
                       ((0, NP - N), (0, LSP - LS))).reshape(NP * LSP)
    ccp_p = jnp.pad(c_cate_pad.astype(I32), ((0, NP - N), (0, 0)))
    part_flat = jnp.concatenate(
        [jnp.arange(NP, dtype=I32)[:, None], ccp_p], axis=1).reshape(NP * LP)

    s_all, rn_all = _make_k1(NP, SV, D, LS, LSP, CHUNK, B1)(
        scene_emb_w, csp_flat)
    agg_flat = _make_k2(NP, D, LP, NV1, CHUNK)(cate_p, rn_all, part_flat)

    a_in = agg_flat.reshape(NP, DP)[:, :D]
    w1t = agg_w[:, :D].T
    w2t = agg_w[:, D:].T
    b2 = agg_b.reshape(1, D)
    BK3 = 1024
    out = pl.pallas_call(
        _k3_body,
        grid=(NP // BK3,),
        in_specs=[
            pl.BlockSpec((BK3, D), lambda i: (i, 0)),
            pl.BlockSpec((BK3, D), lambda i: (i, 0)),
            pl.BlockSpec((D, D), lambda i: (0, 0)),
            pl.BlockSpec((D, D), lambda i: (0, 0)),
            pl.BlockSpec((1, D), lambda i: (0, 0)),
        ],
        out_specs=pl.BlockSpec((BK3, D), lambda i: (i, 0)),
        out_shape=jax.ShapeDtypeStruct((NP, D), F32),
    )(s_all, a_in, w1t, w2t, b2)
    return out[:N]
